# Initial kernel scaffold; baseline (speedup 1.0000x reference)
import jax
import jax.numpy as jnp
from jax.experimental import pallas as pl


def kernel(words, word_seq_lens, context_emb, chars, char_seq_lens, word_embedding):
    raise NotImplementedError("write your pallas kernel here")


# speedup vs baseline: 1.4963x; 1.4963x over previous
import functools

import jax
import jax.numpy as jnp
from jax import lax
from jax.experimental import pallas as pl
from jax.experimental.pallas import tpu as pltpu
from jax.experimental.pallas import tpu_sc as plsc


@functools.lru_cache(maxsize=None)
def _make_gather(V, D, B):
    info = plsc.get_sparse_core_info()
    NC, NS = info.num_cores, info.num_subcores
    NW = NC * NS
    assert B % NW == 0
    b_per_w = B // NW
    chunk = b_per_w
    for cand in (3200, 1600, 1024, 800, 512, 256, 128, 64, 32, 16, 8):
        if b_per_w % cand == 0:
            chunk = cand
            break
    n_chunks = b_per_w // chunk
    mesh = plsc.VectorSubcoreMesh(core_axis_name="c", subcore_axis_name="s")

    @functools.partial(
        pl.kernel,
        mesh=mesh,
        out_type=jax.ShapeDtypeStruct((B, D), jnp.float32),
        compiler_params=pltpu.CompilerParams(use_tc_tiling_on_sc=False),
        scratch_types=[
            pltpu.VMEM((chunk,), jnp.int32),
            pltpu.VMEM((chunk, D), jnp.float32),
            pltpu.SemaphoreType.DMA,
        ],
    )
    def gather_kernel(table_hbm, idx_hbm, out_hbm, idx_v, rows_v, sem):
        wid = lax.axis_index("s") * NC + lax.axis_index("c")
        w_base = wid * b_per_w

        def body(i, carry):
            base = w_base + i * chunk
            pltpu.sync_copy(idx_hbm.at[pl.ds(base, chunk)], idx_v)
            pltpu.async_copy(table_hbm.at[idx_v], rows_v, sem).wait()
            pltpu.sync_copy(rows_v, out_hbm.at[pl.ds(base, chunk)])
            return carry

        lax.fori_loop(0, n_chunks, body, 0)

    return gather_kernel


def kernel(words, word_seq_lens, context_emb, chars, char_seq_lens, word_embedding):
    B, L = words.shape
    V, D = word_embedding.shape
    idx = words.reshape(B * L)
    out = _make_gather(V, D, B * L)(word_embedding, idx)
    return out.reshape(B, L, D)


# speedup vs baseline: 1.5035x; 1.0048x over previous
import functools

import jax
import jax.numpy as jnp
from jax import lax
from jax.experimental import pallas as pl
from jax.experimental.pallas import tpu as pltpu
from jax.experimental.pallas import tpu_sc as plsc


@functools.lru_cache(maxsize=None)
def _make_gather(V, D, B):
    info = plsc.get_sparse_core_info()
    NC, NS = info.num_cores, info.num_subcores
    NW = NC * NS
    assert B % NW == 0
    b_per_w = B // NW
    chunk = b_per_w
    for cand in (1600, 1280, 1024, 800, 512, 256, 128, 64, 32, 16, 8):
        if b_per_w % cand == 0:
            chunk = cand
            break
    n_chunks = b_per_w // chunk
    mesh = plsc.VectorSubcoreMesh(core_axis_name="c", subcore_axis_name="s")

    @functools.partial(
        pl.kernel,
        mesh=mesh,
        out_type=jax.ShapeDtypeStruct((B, D), jnp.float32),
        compiler_params=pltpu.CompilerParams(use_tc_tiling_on_sc=False),
        scratch_types=[
            pltpu.VMEM((b_per_w,), jnp.int32),
            pltpu.VMEM((2, chunk, D), jnp.float32),
            pltpu.SemaphoreType.DMA((2,)),
            pltpu.SemaphoreType.DMA((2,)),
        ],
    )
    def gather_kernel(table_hbm, idx_hbm, out_hbm, idx_v, rows_v, gsem, ssem):
        wid = lax.axis_index("s") * NC + lax.axis_index("c")
        w_base = wid * b_per_w
        pltpu.sync_copy(idx_hbm.at[pl.ds(w_base, b_per_w)], idx_v)

        gd = [None, None]
        sd = [None, None]
        for i in range(n_chunks):
            b = i % 2
            if sd[b] is not None:
                sd[b].wait()
            gd[b] = pltpu.async_copy(
                table_hbm.at[idx_v.at[pl.ds(i * chunk, chunk)]],
                rows_v.at[b],
                gsem.at[b],
            )
            if i >= 1:
                p = 1 - b
                gd[p].wait()
                sd[p] = pltpu.async_copy(
                    rows_v.at[p],
                    out_hbm.at[pl.ds(w_base + (i - 1) * chunk, chunk)],
                    ssem.at[p],
                )
        last = (n_chunks - 1) % 2
        gd[last].wait()
        sd[last] = pltpu.async_copy(
            rows_v.at[last],
            out_hbm.at[pl.ds(w_base + (n_chunks - 1) * chunk, chunk)],
            ssem.at[last],
        )
        sd[1 - last].wait()
        sd[last].wait()

    return gather_kernel


def kernel(words, word_seq_lens, context_emb, chars, char_seq_lens, word_embedding):
    B, L = words.shape
    V, D = word_embedding.shape
    idx = words.reshape(B * L)
    out = _make_gather(V, D, B * L)(word_embedding, idx)
    return out.reshape(B, L, D)
